# baseline (speedup 1.0000x reference)
import jax
import jax.numpy as jnp
from jax.experimental import pallas as pl
from jax.experimental.pallas import tpu as pltpu
from jax.experimental.pallas import tpu_sc as plsc

_WIN = 128
_BLK = 512
_TS = 8192
_WG = 512


def _tc_format_table(W, V, D):
    WT = jnp.transpose(W)
    nblk = (V + 2 * _WG - 1) // (2 * _WG)
    VP = nblk * 2 * _WG

    def body(in_ref, out_ref):
        t = in_ref[...].T
        out_ref[...] = jnp.concatenate([t[:_WG], t[_WG:]], axis=1)

    out = pl.pallas_call(
        body,
        grid=(nblk,),
        in_specs=[pl.BlockSpec((D, 2 * _WG), lambda g: (0, g))],
        out_specs=pl.BlockSpec((_WG, 2 * D), lambda g: (g, 0)),
        out_shape=jax.ShapeDtypeStruct((VP // 2, 2 * D), jnp.float32),
    )(WT)
    return out.reshape(VP, D), VP


def _sc_gather(idx, Wf, N, VP, D):
    mesh = plsc.VectorSubcoreMesh(core_axis_name="core",
                                  subcore_axis_name="subcore")

    @pl.kernel(out_type=jax.ShapeDtypeStruct((N, D), Wf.dtype), mesh=mesh,
               compiler_params=pltpu.CompilerParams(use_tc_tiling_on_sc=False),
               scratch_types=[pltpu.SemaphoreType.DMA])
    def gather_kernel(w_hbm, i_hbm, o_hbm, sem):
        def body(i_vmem, o_vmem):
            copies = [
                pltpu.async_copy(
                    w_hbm.at[i_vmem.at[0, pl.ds(j * _WIN, _WIN)]],
                    o_vmem.at[pl.ds(j * _WIN, _WIN)],
                    sem,
                )
                for j in range(_BLK // _WIN)
            ]
            for c in copies:
                c.wait()

        pltpu.emit_pipeline(
            body,
            grid=(N // _BLK,),
            in_specs=[pl.BlockSpec((1, _BLK), index_map=lambda i: (0, i))],
            out_specs=[pl.BlockSpec((_BLK, D), index_map=lambda i: (i, 0))],
            core_axis_name=("core", "subcore"),
            dimension_semantics=(pltpu.PARALLEL,),
        )(i_hbm, o_hbm)

    return gather_kernel(Wf, idx)


def _tc_transpose(mid2d, H, B, D):
    mb = B // 2 // _TS
    cb = B // _TS

    def body(in_ref, out_ref):
        p = pl.program_id(0) % cb // mb
        blk = in_ref[...]

        @pl.when(p == 0)
        def _():
            out_ref[0] = blk[:, :D].T

        @pl.when(p == 1)
        def _():
            out_ref[0] = blk[:, D:].T

    return pl.pallas_call(
        body,
        grid=(H * cb,),
        in_specs=[pl.BlockSpec(
            (_TS, 128),
            lambda g: ((g // cb) * mb + g % cb % mb, 0))],
        out_specs=pl.BlockSpec(
            (1, D, _TS), lambda g: (g // cb, 0, g % cb)),
        out_shape=jax.ShapeDtypeStruct((H, D, B), jnp.float32),
    )(mid2d)


def kernel(x, W):
    B, H = x.shape
    V, D = W.shape
    N = B * H

    Wf, VP = _tc_format_table(W, V, D)

    xt = jnp.transpose(x)
    xperm = jnp.transpose(xt.reshape(H, 2, B // 2), (0, 2, 1))
    v = xperm.reshape(1, N)
    G = 2 * _WG
    idx = G * (v // G) + 2 * (v % _WG) + (v % G) // _WG

    mid = _sc_gather(idx, Wf, N, VP, D)
    mid2d = mid.reshape(N * D // 128, 128)
    out3 = _tc_transpose(mid2d, H, B, D)
    return jnp.transpose(out3, (2, 0, 1))


# speedup vs baseline: 1.3109x; 1.3109x over previous
import jax
import jax.numpy as jnp
from jax.experimental import pallas as pl
from jax.experimental.pallas import tpu as pltpu
from jax.experimental.pallas import tpu_sc as plsc

_WIN = 128
_BLK = 512
_TS = 8192


def _sc_gather(idx, W, N, D):
    mesh = plsc.VectorSubcoreMesh(core_axis_name="core",
                                  subcore_axis_name="subcore")

    @pl.kernel(out_type=jax.ShapeDtypeStruct((N, D), W.dtype), mesh=mesh,
               compiler_params=pltpu.CompilerParams(use_tc_tiling_on_sc=False),
               scratch_types=[pltpu.SemaphoreType.DMA])
    def gather_kernel(w_hbm, i_hbm, o_hbm, sem):
        def body(i_vmem, o_vmem):
            copies = [
                pltpu.async_copy(
                    w_hbm.at[i_vmem.at[0, pl.ds(j * _WIN, _WIN)]],
                    o_vmem.at[pl.ds(j * _WIN, _WIN)],
                    sem,
                )
                for j in range(_BLK // _WIN)
            ]
            for c in copies:
                c.wait()

        pltpu.emit_pipeline(
            body,
            grid=(N // _BLK,),
            in_specs=[pl.BlockSpec((1, _BLK), index_map=lambda i: (0, i))],
            out_specs=[pl.BlockSpec((_BLK, D), index_map=lambda i: (i, 0))],
            core_axis_name=("core", "subcore"),
            dimension_semantics=(pltpu.PARALLEL,),
        )(i_hbm, o_hbm)

    return gather_kernel(W, idx)


def _tc_transpose(mid2d, H, B, D):
    mb = B // 2 // _TS
    cb = B // _TS

    def body(in_ref, out_ref):
        p = pl.program_id(0) % cb // mb
        blk = in_ref[...]

        @pl.when(p == 0)
        def _():
            out_ref[0] = blk[:, :D].T

        @pl.when(p == 1)
        def _():
            out_ref[0] = blk[:, D:].T

    return pl.pallas_call(
        body,
        grid=(H * cb,),
        in_specs=[pl.BlockSpec(
            (_TS, 128),
            lambda g: ((g // cb) * mb + g % cb % mb, 0))],
        out_specs=pl.BlockSpec(
            (1, D, _TS), lambda g: (g // cb, 0, g % cb)),
        out_shape=jax.ShapeDtypeStruct((H, D, B), jnp.float32),
    )(mid2d)


def kernel(x, W):
    B, H = x.shape
    V, D = W.shape
    N = B * H

    xt = jnp.transpose(x)
    xperm = jnp.transpose(xt.reshape(H, 2, B // 2), (0, 2, 1))
    idx = xperm.reshape(1, N)

    mid = _sc_gather(idx, W, N, D)
    mid2d = mid.reshape(N * D // 128, 128)
    out3 = _tc_transpose(mid2d, H, B, D)
    return jnp.transpose(out3, (2, 0, 1))


# speedup vs baseline: 1.3513x; 1.0308x over previous
import jax
import jax.numpy as jnp
from jax.experimental import pallas as pl
from jax.experimental.pallas import tpu as pltpu
from jax.experimental.pallas import tpu_sc as plsc

_WIN = 128
_BLK = 512
_TS = 8192


def _sc_gather(idx, W, N, D):
    mesh = plsc.VectorSubcoreMesh(core_axis_name="core",
                                  subcore_axis_name="subcore")

    @pl.kernel(out_type=jax.ShapeDtypeStruct((N, D), W.dtype), mesh=mesh,
               compiler_params=pltpu.CompilerParams(use_tc_tiling_on_sc=False),
               scratch_types=[pltpu.SemaphoreType.DMA])
    def gather_kernel(w_hbm, i_hbm, o_hbm, sem):
        def body(i_vmem, o_vmem):
            copies = [
                pltpu.async_copy(
                    w_hbm.at[i_vmem.at[0, pl.ds(j * _WIN, _WIN)]],
                    o_vmem.at[pl.ds(j * _WIN, _WIN)],
                    sem,
                )
                for j in range(_BLK // _WIN)
            ]
            for c in copies:
                c.wait()

        pltpu.emit_pipeline(
            body,
            grid=(N // _BLK,),
            in_specs=[pl.BlockSpec((1, _BLK), index_map=lambda i: (0, i))],
            out_specs=[pl.BlockSpec((_BLK, D), index_map=lambda i: (i, 0))],
            core_axis_name=("core", "subcore"),
            dimension_semantics=(pltpu.PARALLEL,),
        )(i_hbm, o_hbm)

    return gather_kernel(W, idx)


def _tc_transpose(mid2d, H, B, D):
    def body(in_ref, out_ref):
        blk = in_ref[...]
        out_ref[0] = jnp.concatenate([blk[:, :D].T, blk[:, D:].T], axis=1)

    return pl.pallas_call(
        body,
        grid=(H,),
        in_specs=[pl.BlockSpec((B // 2, 128), lambda g: (g, 0))],
        out_specs=pl.BlockSpec((1, D, B), lambda g: (g, 0, 0)),
        out_shape=jax.ShapeDtypeStruct((H, D, B), jnp.float32),
    )(mid2d)


def kernel(x, W):
    B, H = x.shape
    V, D = W.shape
    N = B * H

    xt = jnp.transpose(x)
    xperm = jnp.transpose(xt.reshape(H, 2, B // 2), (0, 2, 1))
    idx = xperm.reshape(1, N)

    mid = _sc_gather(idx, W, N, D)
    mid2d = mid.reshape(N * D // 128, 128)
    out3 = _tc_transpose(mid2d, H, B, D)
    return jnp.transpose(out3, (2, 0, 1))
